# baseline (speedup 1.0000x reference)
import functools

import jax
import jax.numpy as jnp
from jax import lax
from jax.experimental import pallas as pl
from jax.experimental.pallas import tpu as pltpu
from jax.experimental.pallas import tpu_sc as plsc

N_NODES = 10000
N_EDGES = 320000
D = 128
N_GRAPHS = 256

NC = 2
NS = 16
NW = NC * NS

CHUNK = 128
NCH = 80
E_PER_W = NCH * CHUNK
E_PAD = NW * E_PER_W
NCH_STAGE = 40
ROWS_PER_TILE = 632
N_PAD = NS * ROWS_PER_TILE


def _sc_edge_agg_body(src_hbm, dst_hbm, x_hbm, zero_hbm, out_hbm,
                      idx_s, idx_d, rows0, rows1, acc, sem_g, sem_s, sem_i):
    cid = lax.axis_index("c")
    sid = lax.axis_index("s")
    wid = cid * NS + sid
    r0 = sid * ROWS_PER_TILE

    pltpu.async_copy(zero_hbm.at[pl.ds(r0, ROWS_PER_TILE)],
                     acc.at[pl.ds(r0, ROWS_PER_TILE)], sem_i)
    pltpu.async_copy(dst_hbm.at[pl.ds(wid * NCH, NCH)], idx_d, sem_i)
    pltpu.async_copy(src_hbm.at[pl.ds(wid * NCH, NCH_STAGE)], idx_s, sem_i)
    pltpu.make_async_copy(zero_hbm.at[pl.ds(r0, ROWS_PER_TILE)],
                          acc.at[pl.ds(r0, ROWS_PER_TILE)], sem_i).wait()
    pltpu.make_async_copy(dst_hbm.at[pl.ds(wid * NCH, NCH)], idx_d,
                          sem_i).wait()
    pltpu.make_async_copy(src_hbm.at[pl.ds(wid * NCH, NCH_STAGE)], idx_s,
                          sem_i).wait()

    plsc.subcore_barrier()

    def wait_gather(buf):
        pltpu.make_async_copy(x_hbm.at[idx_s.at[0]], buf, sem_g).wait()

    def wait_scatter(buf):
        pltpu.make_async_copy(x_hbm.at[idx_s.at[0]], buf, sem_s).wait()

    pltpu.async_copy(x_hbm.at[idx_s.at[0]], rows0, sem_g)
    pltpu.async_copy(x_hbm.at[idx_s.at[1]], rows1, sem_g)
    for ph in range(NCH // NCH_STAGE):
        dbase = ph * NCH_STAGE

        @pl.loop(0, NCH_STAGE, step=2)
        def _chunk(ci):
            wait_gather(rows0)
            pltpu.async_copy(rows0, acc.at[idx_d.at[dbase + ci]], sem_s,
                             add=True)
            wait_gather(rows1)
            pltpu.async_copy(rows1, acc.at[idx_d.at[dbase + ci + 1]], sem_s,
                             add=True)

            @pl.when(ci + 2 < NCH_STAGE)
            def _():
                wait_scatter(rows0)
                pltpu.async_copy(x_hbm.at[idx_s.at[ci + 2]], rows0, sem_g)
                wait_scatter(rows1)
                pltpu.async_copy(x_hbm.at[idx_s.at[ci + 3]], rows1, sem_g)

        if ph + 1 < NCH // NCH_STAGE:
            pltpu.sync_copy(
                src_hbm.at[pl.ds(wid * NCH + (ph + 1) * NCH_STAGE,
                                 NCH_STAGE)], idx_s)
            wait_scatter(rows0)
            pltpu.async_copy(x_hbm.at[idx_s.at[0]], rows0, sem_g)
            wait_scatter(rows1)
            pltpu.async_copy(x_hbm.at[idx_s.at[1]], rows1, sem_g)
        else:
            wait_scatter(rows0)
            wait_scatter(rows1)

    plsc.subcore_barrier()

    pltpu.sync_copy(acc.at[pl.ds(r0, ROWS_PER_TILE)],
                    out_hbm.at[pl.ds(cid * N_PAD + r0, ROWS_PER_TILE)])


@functools.cache
def _sc_edge_agg():
    return functools.partial(
        pl.kernel,
        out_type=jax.ShapeDtypeStruct((NC * N_PAD, D), jnp.float32),
        mesh=plsc.VectorSubcoreMesh(core_axis_name="c", subcore_axis_name="s",
                                    num_cores=NC, num_subcores=NS),
        scratch_types=[
            pltpu.VMEM((NCH_STAGE, CHUNK), jnp.int32),
            pltpu.VMEM((NCH, CHUNK), jnp.int32),
            pltpu.VMEM((CHUNK, D), jnp.float32),
            pltpu.VMEM((CHUNK, D), jnp.float32),
            pltpu.VMEM_SHARED((N_PAD, D), jnp.float32),
            pltpu.SemaphoreType.DMA,
            pltpu.SemaphoreType.DMA,
            pltpu.SemaphoreType.DMA,
        ],
    )(_sc_edge_agg_body)


BLK = 1000
GRID = N_NODES // BLK


def _tc_fused_body(x_ref, agg_ref, batch_ref, Wm_ref, bm_ref,
                   W1_ref, b1_ref, W2_ref, b2_ref, out_ref, sums, counts):
    i = pl.program_id(0)

    @pl.when(i == 0)
    def _():
        sums[...] = jnp.zeros_like(sums)
        counts[...] = jnp.zeros_like(counts)

    xa = x_ref[...] + agg_ref[0] + agg_ref[1]
    hv = jnp.dot(xa, Wm_ref[...], preferred_element_type=jnp.float32)
    hv = jnp.maximum(hv + bm_ref[...], 0.0)

    seg = batch_ref[0]
    onehot = (lax.broadcasted_iota(jnp.int32, (N_GRAPHS, BLK), 0)
              == seg).astype(jnp.float32)
    sums[...] += jnp.dot(onehot, hv, preferred_element_type=jnp.float32)
    counts[...] += jnp.broadcast_to(
        jnp.sum(onehot, axis=1, keepdims=True), (N_GRAPHS, D))

    @pl.when(i == pl.num_programs(0) - 1)
    def _():
        H = sums[...] / jnp.maximum(counts[...], 1.0)
        h1 = jnp.dot(H, W1_ref[...], preferred_element_type=jnp.float32)
        h1 = jnp.maximum(h1 + b1_ref[...], 0.0)
        Z = jnp.dot(h1, W2_ref[...], preferred_element_type=jnp.float32)
        out_ref[...] = Z + b2_ref[...] + H


_tc_fused = pl.pallas_call(
    _tc_fused_body,
    grid=(GRID,),
    in_specs=[
        pl.BlockSpec((BLK, D), lambda i: (i, 0)),
        pl.BlockSpec((NC, BLK, D), lambda i: (0, i, 0)),
        pl.BlockSpec((1, 1, BLK), lambda i: (i, 0, 0)),
        pl.BlockSpec((D, D), lambda i: (0, 0)),
        pl.BlockSpec((1, D), lambda i: (0, 0)),
        pl.BlockSpec((D, D), lambda i: (0, 0)),
        pl.BlockSpec((1, D), lambda i: (0, 0)),
        pl.BlockSpec((D, D), lambda i: (0, 0)),
        pl.BlockSpec((1, D), lambda i: (0, 0)),
    ],
    out_specs=pl.BlockSpec((N_GRAPHS, D), lambda i: (0, 0)),
    out_shape=jax.ShapeDtypeStruct((N_GRAPHS, D), jnp.float32),
    scratch_shapes=[
        pltpu.VMEM((N_GRAPHS, D), jnp.float32),
        pltpu.VMEM((N_GRAPHS, D), jnp.float32),
    ],
)


def kernel(x, edge_index, batch, W_msg, b_msg, W1, b1, W2, b2):
    src = edge_index[0].astype(jnp.int32)
    dst = edge_index[1].astype(jnp.int32)
    pad = E_PAD - N_EDGES
    ar = jnp.arange(pad, dtype=jnp.int32)
    src = jnp.concatenate([src, ar % N_NODES])
    dst = jnp.concatenate([dst, N_NODES + ar % (N_PAD - N_NODES)])
    src2d = src.reshape(E_PAD // CHUNK, CHUNK)
    dst2d = dst.reshape(E_PAD // CHUNK, CHUNK)
    zeros = jnp.zeros((N_PAD, D), jnp.float32)

    agg = _sc_edge_agg()(src2d, dst2d, x, zeros)
    agg = agg.reshape(NC, N_PAD, D)

    batch3 = batch.astype(jnp.int32).reshape(GRID, 1, BLK)
    bm = b_msg.reshape(1, D)
    b1r = b1.reshape(1, D)
    b2r = b2.reshape(1, D)
    return _tc_fused(x, agg, batch3, W_msg, bm, W1, b1r, W2, b2r)


# speedup vs baseline: 1.0310x; 1.0310x over previous
import functools

import jax
import jax.numpy as jnp
from jax import lax
from jax.experimental import pallas as pl
from jax.experimental.pallas import tpu as pltpu
from jax.experimental.pallas import tpu_sc as plsc

N_NODES = 10000
N_EDGES = 320000
D = 128
N_GRAPHS = 256

NC = 2
NS = 16
NW = NC * NS

CHUNK = 128
NCH = 80
NCH_TAIL = 20
NCH_STAGE = 40
ROWS_PER_TILE = 632
N_PAD = NS * ROWS_PER_TILE


def _sc_edge_agg_body(ei_hbm, x_hbm, zero_hbm, out_hbm,
                      idx, rows0, rows1, acc, sem_g, sem_s, sem_i):
    cid = lax.axis_index("c")
    sid = lax.axis_index("s")
    wid = cid * NS + sid
    r0 = sid * ROWS_PER_TILE

    ibase = 2 * wid * NCH
    is_full = wid < NW - 1
    nrows0 = jnp.where(is_full, 2 * NCH_STAGE, 2 * NCH_TAIL)

    pltpu.async_copy(zero_hbm, acc.at[pl.ds(r0, ROWS_PER_TILE)], sem_i)

    @pl.when(is_full)
    def _():
        pltpu.async_copy(ei_hbm.at[pl.ds(ibase, 2 * NCH_STAGE)], idx, sem_i)
        pltpu.make_async_copy(ei_hbm.at[pl.ds(ibase, 2 * NCH_STAGE)], idx,
                              sem_i).wait()

    @pl.when(jnp.logical_not(is_full))
    def _():
        pltpu.async_copy(ei_hbm.at[pl.ds(ibase, 2 * NCH_TAIL)],
                         idx.at[pl.ds(0, 2 * NCH_TAIL)], sem_i)
        pltpu.make_async_copy(ei_hbm.at[pl.ds(ibase, 2 * NCH_TAIL)],
                              idx.at[pl.ds(0, 2 * NCH_TAIL)], sem_i).wait()

    pltpu.make_async_copy(zero_hbm,
                          acc.at[pl.ds(r0, ROWS_PER_TILE)], sem_i).wait()

    plsc.subcore_barrier()

    def wait_gather(buf):
        pltpu.make_async_copy(x_hbm.at[idx.at[0]], buf, sem_g).wait()

    def wait_scatter(buf):
        pltpu.make_async_copy(x_hbm.at[idx.at[0]], buf, sem_s).wait()

    pltpu.async_copy(x_hbm.at[idx.at[0]], rows0, sem_g)
    pltpu.async_copy(x_hbm.at[idx.at[2]], rows1, sem_g)

    @pl.loop(0, nrows0, step=4)
    def _chunk(ci):
        wait_gather(rows0)
        pltpu.async_copy(rows0, acc.at[idx.at[ci + 1]], sem_s, add=True)
        wait_gather(rows1)
        pltpu.async_copy(rows1, acc.at[idx.at[ci + 3]], sem_s, add=True)

        @pl.when(ci + 4 < nrows0)
        def _():
            wait_scatter(rows0)
            pltpu.async_copy(x_hbm.at[idx.at[ci + 4]], rows0, sem_g)
            wait_scatter(rows1)
            pltpu.async_copy(x_hbm.at[idx.at[ci + 6]], rows1, sem_g)

    wait_scatter(rows0)
    wait_scatter(rows1)

    @pl.when(is_full)
    def _():
        pltpu.sync_copy(
            ei_hbm.at[pl.ds(ibase + 2 * NCH_STAGE, 2 * NCH_STAGE)], idx)
        pltpu.async_copy(x_hbm.at[idx.at[0]], rows0, sem_g)
        pltpu.async_copy(x_hbm.at[idx.at[2]], rows1, sem_g)

        @pl.loop(0, 2 * NCH_STAGE, step=4)
        def _chunk2(ci):
            wait_gather(rows0)
            pltpu.async_copy(rows0, acc.at[idx.at[ci + 1]], sem_s, add=True)
            wait_gather(rows1)
            pltpu.async_copy(rows1, acc.at[idx.at[ci + 3]], sem_s, add=True)

            @pl.when(ci + 4 < 2 * NCH_STAGE)
            def _():
                wait_scatter(rows0)
                pltpu.async_copy(x_hbm.at[idx.at[ci + 4]], rows0, sem_g)
                wait_scatter(rows1)
                pltpu.async_copy(x_hbm.at[idx.at[ci + 6]], rows1, sem_g)

        wait_scatter(rows0)
        wait_scatter(rows1)

    plsc.subcore_barrier()

    pltpu.sync_copy(acc.at[pl.ds(r0, ROWS_PER_TILE)],
                    out_hbm.at[pl.ds(cid * N_PAD + r0, ROWS_PER_TILE)])


@functools.cache
def _sc_edge_agg():
    return functools.partial(
        pl.kernel,
        out_type=jax.ShapeDtypeStruct((NC * N_PAD, D), jnp.float32),
        mesh=plsc.VectorSubcoreMesh(core_axis_name="c", subcore_axis_name="s",
                                    num_cores=NC, num_subcores=NS),
        scratch_types=[
            pltpu.VMEM((2 * NCH_STAGE, CHUNK), jnp.int32),
            pltpu.VMEM((CHUNK, D), jnp.float32),
            pltpu.VMEM((CHUNK, D), jnp.float32),
            pltpu.VMEM_SHARED((N_PAD, D), jnp.float32),
            pltpu.SemaphoreType.DMA,
            pltpu.SemaphoreType.DMA,
            pltpu.SemaphoreType.DMA,
        ],
    )(_sc_edge_agg_body)


BLK = 2000
GRID = N_NODES // BLK


def _tc_fused_body(x_ref, agg_ref, batch_ref, Wm_ref, bm_ref,
                   W1_ref, b1_ref, W2_ref, b2_ref, out_ref, sums, counts):
    i = pl.program_id(0)

    @pl.when(i == 0)
    def _():
        sums[...] = jnp.zeros_like(sums)
        counts[...] = jnp.zeros_like(counts)

    xa = x_ref[...] + agg_ref[0] + agg_ref[1]
    hv = jnp.dot(xa, Wm_ref[...], preferred_element_type=jnp.float32)
    hv = jnp.maximum(hv + bm_ref[...], 0.0)

    seg = batch_ref[0]
    onehot = (lax.broadcasted_iota(jnp.int32, (N_GRAPHS, BLK), 0)
              == seg).astype(jnp.float32)
    sums[...] += jnp.dot(onehot, hv, preferred_element_type=jnp.float32)
    counts[...] += jnp.broadcast_to(
        jnp.sum(onehot, axis=1, keepdims=True), (N_GRAPHS, D))

    @pl.when(i == pl.num_programs(0) - 1)
    def _():
        H = sums[...] / jnp.maximum(counts[...], 1.0)
        h1 = jnp.dot(H, W1_ref[...], preferred_element_type=jnp.float32)
        h1 = jnp.maximum(h1 + b1_ref[...], 0.0)
        Z = jnp.dot(h1, W2_ref[...], preferred_element_type=jnp.float32)
        out_ref[...] = Z + b2_ref[...] + H


_tc_fused = pl.pallas_call(
    _tc_fused_body,
    grid=(GRID,),
    in_specs=[
        pl.BlockSpec((BLK, D), lambda i: (i, 0)),
        pl.BlockSpec((NC, BLK, D), lambda i: (0, i, 0)),
        pl.BlockSpec((1, 1, BLK), lambda i: (i, 0, 0)),
        pl.BlockSpec((D, D), lambda i: (0, 0)),
        pl.BlockSpec((1, D), lambda i: (0, 0)),
        pl.BlockSpec((D, D), lambda i: (0, 0)),
        pl.BlockSpec((1, D), lambda i: (0, 0)),
        pl.BlockSpec((D, D), lambda i: (0, 0)),
        pl.BlockSpec((1, D), lambda i: (0, 0)),
    ],
    out_specs=pl.BlockSpec((N_GRAPHS, D), lambda i: (0, 0)),
    out_shape=jax.ShapeDtypeStruct((N_GRAPHS, D), jnp.float32),
    scratch_shapes=[
        pltpu.VMEM((N_GRAPHS, D), jnp.float32),
        pltpu.VMEM((N_GRAPHS, D), jnp.float32),
    ],
)


def kernel(x, edge_index, batch, W_msg, b_msg, W1, b1, W2, b2):
    ei = edge_index.astype(jnp.int32)
    ei_rows = ei.reshape(2, N_EDGES // CHUNK, CHUNK).transpose(1, 0, 2)
    ei_rows = ei_rows.reshape(2 * N_EDGES // CHUNK, CHUNK)
    zeros = jnp.zeros((ROWS_PER_TILE, D), jnp.float32)

    agg = _sc_edge_agg()(ei_rows, x, zeros)
    agg = agg.reshape(NC, N_PAD, D)

    batch3 = batch.astype(jnp.int32).reshape(GRID, 1, BLK)
    bm = b_msg.reshape(1, D)
    b1r = b1.reshape(1, D)
    b2r = b2.reshape(1, D)
    return _tc_fused(x, agg, batch3, W_msg, bm, W1, b1r, W2, b2r)


# speedup vs baseline: 1.0705x; 1.0383x over previous
import functools

import jax
import jax.numpy as jnp
import numpy as np
from jax import lax
from jax.experimental import pallas as pl
from jax.experimental.pallas import tpu as pltpu
from jax.experimental.pallas import tpu_sc as plsc

N_NODES = 10000
N_EDGES = 320000
D = 128
N_GRAPHS = 256

NC = 2
NS = 16
NW = NC * NS

CHUNK = 128
NCH = 80
E_PER_W = NCH * CHUNK
E_PAD = NW * E_PER_W
NCH_STAGE = 40
ROWS_PER_TILE = 632
N_PAD = NS * ROWS_PER_TILE

_PAD_AR = np.arange(E_PAD - N_EDGES, dtype=np.int32)
_PAD_EDGES = np.stack([_PAD_AR % N_NODES,
                       N_NODES + _PAD_AR % (N_PAD - N_NODES)])


def _sc_edge_agg_body(ei_hbm, x_hbm, zero_hbm, out_hbm,
                      idx_s, idx_d, rows0, rows1, acc, sem_g, sem_s, sem_i):
    cid = lax.axis_index("c")
    sid = lax.axis_index("s")
    wid = cid * NS + sid
    r0 = sid * ROWS_PER_TILE

    pltpu.async_copy(zero_hbm, acc.at[pl.ds(r0, ROWS_PER_TILE)], sem_i)
    pltpu.async_copy(ei_hbm.at[1, pl.ds(wid * NCH, NCH)], idx_d, sem_i)
    pltpu.async_copy(ei_hbm.at[0, pl.ds(wid * NCH, NCH_STAGE)], idx_s, sem_i)
    pltpu.make_async_copy(zero_hbm,
                          acc.at[pl.ds(r0, ROWS_PER_TILE)], sem_i).wait()
    pltpu.make_async_copy(ei_hbm.at[1, pl.ds(wid * NCH, NCH)], idx_d,
                          sem_i).wait()
    pltpu.make_async_copy(ei_hbm.at[0, pl.ds(wid * NCH, NCH_STAGE)], idx_s,
                          sem_i).wait()

    plsc.subcore_barrier()

    def wait_gather(buf):
        pltpu.make_async_copy(x_hbm.at[idx_s.at[0]], buf, sem_g).wait()

    def wait_scatter(buf):
        pltpu.make_async_copy(x_hbm.at[idx_s.at[0]], buf, sem_s).wait()

    pltpu.async_copy(x_hbm.at[idx_s.at[0]], rows0, sem_g)
    pltpu.async_copy(x_hbm.at[idx_s.at[1]], rows1, sem_g)
    for ph in range(NCH // NCH_STAGE):
        dbase = ph * NCH_STAGE

        @pl.loop(0, NCH_STAGE, step=2)
        def _chunk(ci):
            wait_gather(rows0)
            pltpu.async_copy(rows0, acc.at[idx_d.at[dbase + ci]], sem_s,
                             add=True)
            wait_gather(rows1)
            pltpu.async_copy(rows1, acc.at[idx_d.at[dbase + ci + 1]], sem_s,
                             add=True)

            @pl.when(ci + 2 < NCH_STAGE)
            def _():
                wait_scatter(rows0)
                pltpu.async_copy(x_hbm.at[idx_s.at[ci + 2]], rows0, sem_g)
                wait_scatter(rows1)
                pltpu.async_copy(x_hbm.at[idx_s.at[ci + 3]], rows1, sem_g)

        if ph + 1 < NCH // NCH_STAGE:
            pltpu.sync_copy(
                ei_hbm.at[0, pl.ds(wid * NCH + (ph + 1) * NCH_STAGE,
                                   NCH_STAGE)], idx_s)
            wait_scatter(rows0)
            pltpu.async_copy(x_hbm.at[idx_s.at[0]], rows0, sem_g)
            wait_scatter(rows1)
            pltpu.async_copy(x_hbm.at[idx_s.at[1]], rows1, sem_g)
        else:
            wait_scatter(rows0)
            wait_scatter(rows1)

    plsc.subcore_barrier()

    pltpu.sync_copy(acc.at[pl.ds(r0, ROWS_PER_TILE)],
                    out_hbm.at[pl.ds(cid * N_PAD + r0, ROWS_PER_TILE)])


@functools.cache
def _sc_edge_agg():
    return functools.partial(
        pl.kernel,
        out_type=jax.ShapeDtypeStruct((NC * N_PAD, D), jnp.float32),
        mesh=plsc.VectorSubcoreMesh(core_axis_name="c", subcore_axis_name="s",
                                    num_cores=NC, num_subcores=NS),
        scratch_types=[
            pltpu.VMEM((NCH_STAGE, CHUNK), jnp.int32),
            pltpu.VMEM((NCH, CHUNK), jnp.int32),
            pltpu.VMEM((CHUNK, D), jnp.float32),
            pltpu.VMEM((CHUNK, D), jnp.float32),
            pltpu.VMEM_SHARED((N_PAD, D), jnp.float32),
            pltpu.SemaphoreType.DMA,
            pltpu.SemaphoreType.DMA,
            pltpu.SemaphoreType.DMA,
        ],
    )(_sc_edge_agg_body)


BLK = 2000
GRID = N_NODES // BLK


def _tc_fused_body(x_ref, agg_ref, batch_ref, Wm_ref, bm_ref,
                   W1_ref, b1_ref, W2_ref, b2_ref, out_ref, sums, counts):
    i = pl.program_id(0)

    @pl.when(i == 0)
    def _():
        sums[...] = jnp.zeros_like(sums)
        counts[...] = jnp.zeros_like(counts)

    xa = x_ref[...] + agg_ref[0] + agg_ref[1]
    hv = jnp.dot(xa, Wm_ref[...], preferred_element_type=jnp.float32)
    hv = jnp.maximum(hv + bm_ref[...], 0.0)

    seg = batch_ref[0]
    onehot = (lax.broadcasted_iota(jnp.int32, (N_GRAPHS, BLK), 0)
              == seg).astype(jnp.float32)
    sums[...] += jnp.dot(onehot, hv, preferred_element_type=jnp.float32)
    counts[...] += jnp.broadcast_to(
        jnp.sum(onehot, axis=1, keepdims=True), (N_GRAPHS, D))

    @pl.when(i == pl.num_programs(0) - 1)
    def _():
        H = sums[...] / jnp.maximum(counts[...], 1.0)
        h1 = jnp.dot(H, W1_ref[...], preferred_element_type=jnp.float32)
        h1 = jnp.maximum(h1 + b1_ref[...], 0.0)
        Z = jnp.dot(h1, W2_ref[...], preferred_element_type=jnp.float32)
        out_ref[...] = Z + b2_ref[...] + H


_tc_fused = pl.pallas_call(
    _tc_fused_body,
    grid=(GRID,),
    in_specs=[
        pl.BlockSpec((BLK, D), lambda i: (i, 0)),
        pl.BlockSpec((NC, BLK, D), lambda i: (0, i, 0)),
        pl.BlockSpec((1, 1, BLK), lambda i: (i, 0, 0)),
        pl.BlockSpec((D, D), lambda i: (0, 0)),
        pl.BlockSpec((1, D), lambda i: (0, 0)),
        pl.BlockSpec((D, D), lambda i: (0, 0)),
        pl.BlockSpec((1, D), lambda i: (0, 0)),
        pl.BlockSpec((D, D), lambda i: (0, 0)),
        pl.BlockSpec((1, D), lambda i: (0, 0)),
    ],
    out_specs=pl.BlockSpec((N_GRAPHS, D), lambda i: (0, 0)),
    out_shape=jax.ShapeDtypeStruct((N_GRAPHS, D), jnp.float32),
    scratch_shapes=[
        pltpu.VMEM((N_GRAPHS, D), jnp.float32),
        pltpu.VMEM((N_GRAPHS, D), jnp.float32),
    ],
)


def kernel(x, edge_index, batch, W_msg, b_msg, W1, b1, W2, b2):
    ei = jnp.concatenate([edge_index.astype(jnp.int32), _PAD_EDGES], axis=1)
    ei3 = ei.reshape(2, E_PAD // CHUNK, CHUNK)
    zeros = jnp.zeros((ROWS_PER_TILE, D), jnp.float32)

    agg = _sc_edge_agg()(ei3, x, zeros)
    agg = agg.reshape(NC, N_PAD, D)

    batch3 = batch.astype(jnp.int32).reshape(GRID, 1, BLK)
    bm = b_msg.reshape(1, D)
    b1r = b1.reshape(1, D)
    b2r = b2.reshape(1, D)
    return _tc_fused(x, agg, batch3, W_msg, bm, W1, b1r, W2, b2r)


# speedup vs baseline: 1.0742x; 1.0034x over previous
import functools

import jax
import jax.numpy as jnp
import numpy as np
from jax import lax
from jax.experimental import pallas as pl
from jax.experimental.pallas import tpu as pltpu
from jax.experimental.pallas import tpu_sc as plsc

N_NODES = 10000
N_EDGES = 320000
D = 128
N_GRAPHS = 256

NC = 2
NS = 16
NW = NC * NS

CHUNK = 128
NCH = 80
E_PER_W = NCH * CHUNK
E_PAD = NW * E_PER_W
NCH_STAGE = 40
ROWS_PER_TILE = 632
N_PAD = NS * ROWS_PER_TILE

_PAD_AR = np.arange(E_PAD - N_EDGES, dtype=np.int32)
_PAD_EDGES = np.stack([_PAD_AR % N_NODES,
                       N_NODES + _PAD_AR % (N_PAD - N_NODES)])


def _sc_edge_agg_body(ei_hbm, x_hbm, zero_hbm, out_hbm,
                      idx_s, idx_d, rows0, rows1, acc, sem_g, sem_s, sem_i):
    cid = lax.axis_index("c")
    sid = lax.axis_index("s")
    wid = cid * NS + sid
    r0 = sid * ROWS_PER_TILE

    pltpu.async_copy(zero_hbm, acc.at[pl.ds(r0, ROWS_PER_TILE)], sem_i)
    pltpu.async_copy(ei_hbm.at[1, pl.ds(wid * NCH, NCH)], idx_d, sem_i)
    pltpu.async_copy(ei_hbm.at[0, pl.ds(wid * NCH, NCH_STAGE)], idx_s, sem_i)
    pltpu.make_async_copy(zero_hbm,
                          acc.at[pl.ds(r0, ROWS_PER_TILE)], sem_i).wait()
    pltpu.make_async_copy(ei_hbm.at[1, pl.ds(wid * NCH, NCH)], idx_d,
                          sem_i).wait()
    pltpu.make_async_copy(ei_hbm.at[0, pl.ds(wid * NCH, NCH_STAGE)], idx_s,
                          sem_i).wait()

    plsc.subcore_barrier()

    def wait_gather(buf):
        pltpu.make_async_copy(x_hbm.at[idx_s.at[0]], buf, sem_g).wait()

    def wait_scatter(buf):
        pltpu.make_async_copy(x_hbm.at[idx_s.at[0]], buf, sem_s).wait()

    pltpu.async_copy(x_hbm.at[idx_s.at[0]], rows0, sem_g)
    pltpu.async_copy(x_hbm.at[idx_s.at[1]], rows1, sem_g)
    for ph in range(NCH // NCH_STAGE):
        dbase = ph * NCH_STAGE

        @pl.loop(0, NCH_STAGE, step=2)
        def _chunk(ci):
            wait_gather(rows0)
            pltpu.async_copy(rows0, acc.at[idx_d.at[dbase + ci]], sem_s,
                             add=True)
            wait_gather(rows1)
            pltpu.async_copy(rows1, acc.at[idx_d.at[dbase + ci + 1]], sem_s,
                             add=True)

            @pl.when(ci + 2 < NCH_STAGE)
            def _():
                wait_scatter(rows0)
                pltpu.async_copy(x_hbm.at[idx_s.at[ci + 2]], rows0, sem_g)
                wait_scatter(rows1)
                pltpu.async_copy(x_hbm.at[idx_s.at[ci + 3]], rows1, sem_g)

        if ph + 1 < NCH // NCH_STAGE:
            pltpu.sync_copy(
                ei_hbm.at[0, pl.ds(wid * NCH + (ph + 1) * NCH_STAGE,
                                   NCH_STAGE)], idx_s)
            wait_scatter(rows0)
            pltpu.async_copy(x_hbm.at[idx_s.at[0]], rows0, sem_g)
            wait_scatter(rows1)
            pltpu.async_copy(x_hbm.at[idx_s.at[1]], rows1, sem_g)
        else:
            wait_scatter(rows0)
            wait_scatter(rows1)

    plsc.subcore_barrier()

    pltpu.sync_copy(acc.at[pl.ds(r0, ROWS_PER_TILE)],
                    out_hbm.at[pl.ds(cid * N_PAD + r0, ROWS_PER_TILE)])


@functools.cache
def _sc_edge_agg():
    return functools.partial(
        pl.kernel,
        out_type=jax.ShapeDtypeStruct((NC * N_PAD, D), jnp.float32),
        mesh=plsc.VectorSubcoreMesh(core_axis_name="c", subcore_axis_name="s",
                                    num_cores=NC, num_subcores=NS),
        scratch_types=[
            pltpu.VMEM((NCH_STAGE, CHUNK), jnp.int32),
            pltpu.VMEM((NCH, CHUNK), jnp.int32),
            pltpu.VMEM((CHUNK, D), jnp.float32),
            pltpu.VMEM((CHUNK, D), jnp.float32),
            pltpu.VMEM_SHARED((N_PAD, D), jnp.float32),
            pltpu.SemaphoreType.DMA,
            pltpu.SemaphoreType.DMA,
            pltpu.SemaphoreType.DMA,
        ],
    )(_sc_edge_agg_body)


BLK = 5000
GRID = N_NODES // BLK


def _tc_fused_body(x_ref, agg_ref, batch_ref, Wm_ref, bm_ref,
                   W1_ref, b1_ref, W2_ref, b2_ref, out_ref, sums, counts):
    i = pl.program_id(0)

    @pl.when(i == 0)
    def _():
        sums[...] = jnp.zeros_like(sums)
        counts[...] = jnp.zeros_like(counts)

    xa = x_ref[...] + agg_ref[0] + agg_ref[1]
    hv = jnp.dot(xa, Wm_ref[...], preferred_element_type=jnp.float32)
    hv = jnp.maximum(hv + bm_ref[...], 0.0)

    seg = batch_ref[0]
    onehot = (lax.broadcasted_iota(jnp.int32, (N_GRAPHS, BLK), 0)
              == seg).astype(jnp.float32)
    sums[...] += jnp.dot(onehot, hv, preferred_element_type=jnp.float32)
    counts[...] += jnp.broadcast_to(
        jnp.sum(onehot, axis=1, keepdims=True), (N_GRAPHS, D))

    @pl.when(i == pl.num_programs(0) - 1)
    def _():
        H = sums[...] / jnp.maximum(counts[...], 1.0)
        h1 = jnp.dot(H, W1_ref[...], preferred_element_type=jnp.float32)
        h1 = jnp.maximum(h1 + b1_ref[...], 0.0)
        Z = jnp.dot(h1, W2_ref[...], preferred_element_type=jnp.float32)
        out_ref[...] = Z + b2_ref[...] + H


_tc_fused = pl.pallas_call(
    _tc_fused_body,
    grid=(GRID,),
    in_specs=[
        pl.BlockSpec((BLK, D), lambda i: (i, 0)),
        pl.BlockSpec((NC, BLK, D), lambda i: (0, i, 0)),
        pl.BlockSpec((1, 1, BLK), lambda i: (i, 0, 0)),
        pl.BlockSpec((D, D), lambda i: (0, 0)),
        pl.BlockSpec((1, D), lambda i: (0, 0)),
        pl.BlockSpec((D, D), lambda i: (0, 0)),
        pl.BlockSpec((1, D), lambda i: (0, 0)),
        pl.BlockSpec((D, D), lambda i: (0, 0)),
        pl.BlockSpec((1, D), lambda i: (0, 0)),
    ],
    out_specs=pl.BlockSpec((N_GRAPHS, D), lambda i: (0, 0)),
    out_shape=jax.ShapeDtypeStruct((N_GRAPHS, D), jnp.float32),
    scratch_shapes=[
        pltpu.VMEM((N_GRAPHS, D), jnp.float32),
        pltpu.VMEM((N_GRAPHS, D), jnp.float32),
    ],
)


def kernel(x, edge_index, batch, W_msg, b_msg, W1, b1, W2, b2):
    ei = jnp.concatenate([edge_index.astype(jnp.int32), _PAD_EDGES], axis=1)
    ei3 = ei.reshape(2, E_PAD // CHUNK, CHUNK)
    zeros = jnp.zeros((ROWS_PER_TILE, D), jnp.float32)

    agg = _sc_edge_agg()(ei3, x, zeros)
    agg = agg.reshape(NC, N_PAD, D)

    batch3 = batch.astype(jnp.int32).reshape(GRID, 1, BLK)
    bm = b_msg.reshape(1, D)
    b1r = b1.reshape(1, D)
    b2r = b2.reshape(1, D)
    return _tc_fused(x, agg, batch3, W_msg, bm, W1, b1r, W2, b2r)
